# baseline (speedup 1.0000x reference)
import functools

import jax
import jax.numpy as jnp
from jax import lax
from jax.experimental import pallas as pl
from jax.experimental.pallas import tpu as pltpu
from jax.experimental.pallas import tpu_sc as plsc

NUM_PAGES = 2048
PAGE_SIZE = 64
KV_DIM = 256
MAX_SEQS = 64
PAGES_PER_SEQ = 64
MAX_SEQ_LEN = 4096


_BP = 32
_NCH = NUM_PAGES // _BP
_L = 8
_M = 8
_K = _L + _M


def _tc_body(meta_s, seq_lens_s, pi_s, cache_a, cache_out_a,
             bufs, pbuf, in_sems, out_sems, psem):

    def in_copy(c):
        k = jax.lax.rem(c, _K)
        return pltpu.make_async_copy(
            cache_a.at[pl.ds(c * _BP, _BP)], bufs.at[k], in_sems.at[k])

    def out_copy(c):
        k = jax.lax.rem(c, _K)
        return pltpu.make_async_copy(
            bufs.at[k], cache_out_a.at[pl.ds(c * _BP, _BP)], out_sems.at[k])

    for c in range(_L):
        in_copy(c).start()

    def step(c, carry):
        @pl.when(c >= _M)
        def _():
            out_copy(c - _M).wait()

        @pl.when(c + _L < _NCH)
        def _():
            in_copy(c + _L).start()

        in_copy(c).wait()
        out_copy(c).start()
        return carry

    jax.lax.fori_loop(0, _NCH, step, 0)

    parent = meta_s[0]
    fresh = meta_s[1]
    src_len = seq_lens_s[parent]
    safe_last = jnp.maximum((src_len + PAGE_SIZE - 1) // PAGE_SIZE - 1, 0)
    has_partial = jnp.logical_and(src_len % PAGE_SIZE != 0, src_len > 0)
    src_page = pi_s[parent, safe_last]
    dst_page = jnp.where(has_partial, fresh, src_page)

    def drain(c, carry):
        out_copy(c).wait()
        return carry
    jax.lax.fori_loop(max(_NCH - _M, 0), _NCH, drain, 0)

    fin = pltpu.make_async_copy(cache_a.at[pl.ds(src_page, 1)], pbuf, psem)
    fin.start()
    fin.wait()
    fout = pltpu.make_async_copy(pbuf, cache_out_a.at[pl.ds(dst_page, 1)], psem)
    fout.start()
    fout.wait()


def _clone_cache(cache, meta, seq_lens, page_indices):
    return pl.pallas_call(
        _tc_body,
        in_specs=[
            pl.BlockSpec(memory_space=pltpu.SMEM),
            pl.BlockSpec(memory_space=pltpu.SMEM),
            pl.BlockSpec(memory_space=pltpu.SMEM),
            pl.BlockSpec(memory_space=pl.ANY),
        ],
        out_specs=pl.BlockSpec(memory_space=pl.ANY),
        out_shape=jax.ShapeDtypeStruct(cache.shape, cache.dtype),
        scratch_shapes=[
            pltpu.VMEM((_K, _BP, PAGE_SIZE, KV_DIM), jnp.float32),
            pltpu.VMEM((1, PAGE_SIZE, KV_DIM), jnp.float32),
            pltpu.SemaphoreType.DMA((_K,)),
            pltpu.SemaphoreType.DMA((_K,)),
            pltpu.SemaphoreType.DMA,
        ],
        compiler_params=pltpu.CompilerParams(
            vmem_limit_bytes=128 * 1024 * 1024),
    )(meta, seq_lens, page_indices, cache)


_NW = 32
_RPW = MAX_SEQS // _NW


def _sc_body(tokens_h, seq_h, pi_h, meta_h,
             tokens_o, seq_o, pi_o,
             ridx, tbuf, pbuf, piv, mv, sv, sov):
    cid = lax.axis_index("c")
    sid = lax.axis_index("s")
    wid = sid * 2 + cid

    pltpu.sync_copy(meta_h, mv)
    pltpu.sync_copy(seq_h, sv)
    pltpu.sync_copy(pi_h, piv)
    lanes = lax.iota(jnp.int32, 16)
    zeros16 = jnp.zeros((16,), jnp.int32)
    parent_v = plsc.load_gather(mv, [zeros16 + 1])
    child_v = plsc.load_gather(mv, [zeros16 + 2])
    fresh_v = plsc.load_gather(mv, [zeros16 + 3])

    src_len_v = plsc.load_gather(sv, [parent_v])
    last_v = (src_len_v + PAGE_SIZE - 1) // PAGE_SIZE - 1
    safe_last_v = jnp.maximum(last_v, 0)
    partial_v = jnp.logical_and(src_len_v % PAGE_SIZE != 0, src_len_v > 0)

    rows_v = wid * _RPW + lanes
    rsel_v = jnp.where(rows_v == child_v, parent_v, rows_v)
    plsc.store_scatter(ridx, [lanes], rsel_v, mask=lanes < _RPW)
    pltpu.sync_copy(tokens_h.at[ridx], tbuf)
    pltpu.sync_copy(tbuf, tokens_o.at[pl.ds(wid * _RPW, _RPW)])

    for r_off in range(_RPW):
        r = wid * _RPW + r_off
        rsrc_v = jnp.where(r == child_v, parent_v, zeros16 + r)
        is_child_v = jnp.logical_and(r == child_v, partial_v)
        for j in range(PAGES_PER_SEQ // 16):
            v = plsc.load_gather(piv, [rsrc_v, lanes + 16 * j])
            mask = jnp.logical_and(lanes + 16 * j == safe_last_v, is_child_v)
            pbuf[r_off, pl.ds(16 * j, 16)] = jnp.where(mask, fresh_v, v)
    pltpu.sync_copy(pbuf, pi_o.at[pl.ds(wid * _RPW, _RPW)])

    @pl.when(wid == 0)
    def _seq():
        for j in range(MAX_SEQS // 16):
            slj = sv[pl.ds(16 * j, 16)]
            sov[pl.ds(16 * j, 16)] = jnp.where(lanes + 16 * j == child_v,
                                               src_len_v, slj)
        pltpu.sync_copy(sov, seq_o)


def _clone_metadata(tokens, seq_lens, page_indices, meta16):
    mesh = plsc.VectorSubcoreMesh(core_axis_name="c", subcore_axis_name="s")
    sck = functools.partial(
        pl.kernel,
        mesh=mesh,
        compiler_params=pltpu.CompilerParams(needs_layout_passes=False),
        out_type=[
            jax.ShapeDtypeStruct(tokens.shape, tokens.dtype),
            jax.ShapeDtypeStruct(seq_lens.shape, seq_lens.dtype),
            jax.ShapeDtypeStruct(page_indices.shape, page_indices.dtype),
        ],
        scratch_types=[
            pltpu.VMEM((_RPW,), jnp.int32),
            pltpu.VMEM((_RPW, MAX_SEQ_LEN), jnp.int32),
            pltpu.VMEM((_RPW, PAGES_PER_SEQ), jnp.int32),
            pltpu.VMEM((MAX_SEQS, PAGES_PER_SEQ), jnp.int32),
            pltpu.VMEM((16,), jnp.int32),
            pltpu.VMEM((MAX_SEQS,), jnp.int32),
            pltpu.VMEM((MAX_SEQS,), jnp.int32),
        ],
    )(_sc_body)
    return sck(tokens, seq_lens, page_indices, meta16)


def kernel(cache, tokens, seq_lens, page_indices, parent_local_id,
           child_local_id, fresh_page):
    parent = jnp.asarray(parent_local_id, jnp.int32)
    child = jnp.asarray(child_local_id, jnp.int32)
    fresh = jnp.asarray(fresh_page, jnp.int32)
    meta_tc = jnp.stack([parent, fresh])
    meta16 = jnp.pad(jnp.stack([parent, child, fresh]), (1, 12))

    tokens_out, seq_lens_out, pi_out = _clone_metadata(
        tokens, seq_lens, page_indices, meta16)
    cache_out = _clone_cache(cache, meta_tc, seq_lens, page_indices)
    return (cache_out, tokens_out, seq_lens_out, pi_out)


# speedup vs baseline: 1.1816x; 1.1816x over previous
import jax
import jax.numpy as jnp
from jax.experimental import pallas as pl
from jax.experimental.pallas import tpu as pltpu

NUM_PAGES = 2048
PAGE_SIZE = 64
KV_DIM = 256
MAX_SEQS = 64
PAGES_PER_SEQ = 64
MAX_SEQ_LEN = 4096

_BP = 16
_NCH = NUM_PAGES // _BP
_L = 12
_M = 12
_K = _L + _M


def _body(meta_s, seq_lens_s, pi_s, pi_v, tokens_v, cache_a,
          cache_out_a, tokens_out_v, seq_lens_out_s, pi_out_v,
          bufs, pbuf, in_sems, out_sems, psem):

    def in_copy(c):
        k = jax.lax.rem(c, _K)
        return pltpu.make_async_copy(
            cache_a.at[pl.ds(c * _BP, _BP)], bufs.at[k], in_sems.at[k])

    def out_copy(c):
        k = jax.lax.rem(c, _K)
        return pltpu.make_async_copy(
            bufs.at[k], cache_out_a.at[pl.ds(c * _BP, _BP)], out_sems.at[k])

    for c in range(_L):
        in_copy(c).start()

    def step(c, carry):
        @pl.when(c >= _M)
        def _():
            out_copy(c - _M).wait()

        @pl.when(c + _L < _NCH)
        def _():
            in_copy(c + _L).start()

        in_copy(c).wait()
        out_copy(c).start()
        return carry

    jax.lax.fori_loop(0, _NCH, step, 0)

    parent = meta_s[0]
    child = meta_s[1]
    fresh = meta_s[2]

    src_len = seq_lens_s[parent]
    last_idx = (src_len + PAGE_SIZE - 1) // PAGE_SIZE - 1
    safe_last = jnp.maximum(last_idx, 0)
    has_partial = jnp.logical_and(src_len % PAGE_SIZE != 0, src_len > 0)
    src_page = pi_s[parent, safe_last]
    dst_page = jnp.where(has_partial, fresh, src_page)

    rows_t = jax.lax.broadcasted_iota(jnp.int32, (MAX_SEQS, MAX_SEQ_LEN), 0)
    parent_tok = tokens_v[pl.ds(parent, 1), :]
    tokens_out_v[...] = jnp.where(rows_t == child, parent_tok, tokens_v[...])

    rows_p = jax.lax.broadcasted_iota(jnp.int32, (MAX_SEQS, PAGES_PER_SEQ), 0)
    cols_p = jax.lax.broadcasted_iota(jnp.int32, (1, PAGES_PER_SEQ), 1)
    parent_row = pi_v[pl.ds(parent, 1), :]
    child_row = jnp.where(jnp.logical_and(has_partial, cols_p == safe_last),
                          fresh, parent_row)
    pi_out_v[...] = jnp.where(rows_p == child, child_row, pi_v[...])

    def _wr(k, carry):
        v = seq_lens_s[k]
        seq_lens_out_s[k] = jnp.where(k == child, src_len, v)
        return carry
    jax.lax.fori_loop(0, MAX_SEQS, _wr, 0)

    def drain(c, carry):
        out_copy(c).wait()
        return carry
    jax.lax.fori_loop(max(_NCH - _M, 0), _NCH, drain, 0)

    fin = pltpu.make_async_copy(cache_a.at[pl.ds(src_page, 1)], pbuf, psem)
    fin.start()
    fin.wait()
    fout = pltpu.make_async_copy(pbuf, cache_out_a.at[pl.ds(dst_page, 1)], psem)
    fout.start()
    fout.wait()


def kernel(cache, tokens, seq_lens, page_indices, parent_local_id,
           child_local_id, fresh_page):
    meta = jnp.stack([jnp.asarray(parent_local_id, jnp.int32),
                      jnp.asarray(child_local_id, jnp.int32),
                      jnp.asarray(fresh_page, jnp.int32)])
    out = pl.pallas_call(
        _body,
        in_specs=[
            pl.BlockSpec(memory_space=pltpu.SMEM),
            pl.BlockSpec(memory_space=pltpu.SMEM),
            pl.BlockSpec(memory_space=pltpu.SMEM),
            pl.BlockSpec(memory_space=pltpu.VMEM),
            pl.BlockSpec(memory_space=pltpu.VMEM),
            pl.BlockSpec(memory_space=pl.ANY),
        ],
        out_specs=[
            pl.BlockSpec(memory_space=pl.ANY),
            pl.BlockSpec(memory_space=pltpu.VMEM),
            pl.BlockSpec(memory_space=pltpu.SMEM),
            pl.BlockSpec(memory_space=pltpu.VMEM),
        ],
        out_shape=[
            jax.ShapeDtypeStruct(cache.shape, cache.dtype),
            jax.ShapeDtypeStruct(tokens.shape, tokens.dtype),
            jax.ShapeDtypeStruct(seq_lens.shape, seq_lens.dtype),
            jax.ShapeDtypeStruct(page_indices.shape, page_indices.dtype),
        ],
        scratch_shapes=[
            pltpu.VMEM((_K, _BP, PAGE_SIZE, KV_DIM), jnp.float32),
            pltpu.VMEM((1, PAGE_SIZE, KV_DIM), jnp.float32),
            pltpu.SemaphoreType.DMA((_K,)),
            pltpu.SemaphoreType.DMA((_K,)),
            pltpu.SemaphoreType.DMA,
        ],
        compiler_params=pltpu.CompilerParams(
            vmem_limit_bytes=128 * 1024 * 1024),
    )(meta, seq_lens, page_indices, page_indices, tokens, cache)
    cache_out, tokens_out, seq_lens_out, pi_out = out
    return (cache_out, tokens_out, seq_lens_out, pi_out)


# speedup vs baseline: 1.1862x; 1.0039x over previous
import jax
import jax.numpy as jnp
from jax.experimental import pallas as pl
from jax.experimental.pallas import tpu as pltpu

NUM_PAGES = 2048
PAGE_SIZE = 64
KV_DIM = 256
MAX_SEQS = 64
PAGES_PER_SEQ = 64
MAX_SEQ_LEN = 4096

_BP = 64
_NCH = NUM_PAGES // _BP
_L = 5
_M = 5
_K = _L + _M


def _body(meta_s, seq_lens_s, pi_s, pi_v, tokens_v, cache_a,
          cache_out_a, tokens_out_v, seq_lens_out_s, pi_out_v,
          bufs, pbuf, in_sems, out_sems, psem):

    def in_copy(c):
        k = jax.lax.rem(c, _K)
        return pltpu.make_async_copy(
            cache_a.at[pl.ds(c * _BP, _BP)], bufs.at[k], in_sems.at[k])

    def out_copy(c):
        k = jax.lax.rem(c, _K)
        return pltpu.make_async_copy(
            bufs.at[k], cache_out_a.at[pl.ds(c * _BP, _BP)], out_sems.at[k])

    for c in range(_L):
        in_copy(c).start()

    def step(c, carry):
        @pl.when(c >= _M)
        def _():
            out_copy(c - _M).wait()

        @pl.when(c + _L < _NCH)
        def _():
            in_copy(c + _L).start()

        in_copy(c).wait()
        out_copy(c).start()
        return carry

    jax.lax.fori_loop(0, _NCH, step, 0)

    parent = meta_s[0]
    child = meta_s[1]
    fresh = meta_s[2]

    src_len = seq_lens_s[parent]
    last_idx = (src_len + PAGE_SIZE - 1) // PAGE_SIZE - 1
    safe_last = jnp.maximum(last_idx, 0)
    has_partial = jnp.logical_and(src_len % PAGE_SIZE != 0, src_len > 0)
    src_page = pi_s[parent, safe_last]
    dst_page = jnp.where(has_partial, fresh, src_page)

    rows_t = jax.lax.broadcasted_iota(jnp.int32, (MAX_SEQS, MAX_SEQ_LEN), 0)
    parent_tok = tokens_v[pl.ds(parent, 1), :]
    tokens_out_v[...] = jnp.where(rows_t == child, parent_tok, tokens_v[...])

    rows_p = jax.lax.broadcasted_iota(jnp.int32, (MAX_SEQS, PAGES_PER_SEQ), 0)
    cols_p = jax.lax.broadcasted_iota(jnp.int32, (1, PAGES_PER_SEQ), 1)
    parent_row = pi_v[pl.ds(parent, 1), :]
    child_row = jnp.where(jnp.logical_and(has_partial, cols_p == safe_last),
                          fresh, parent_row)
    pi_out_v[...] = jnp.where(rows_p == child, child_row, pi_v[...])

    def _wr(k, carry):
        v = seq_lens_s[k]
        seq_lens_out_s[k] = jnp.where(k == child, src_len, v)
        return carry
    jax.lax.fori_loop(0, MAX_SEQS, _wr, 0)

    def drain(c, carry):
        out_copy(c).wait()
        return carry
    jax.lax.fori_loop(max(_NCH - _M, 0), _NCH, drain, 0)

    fin = pltpu.make_async_copy(cache_a.at[pl.ds(src_page, 1)], pbuf, psem)
    fin.start()
    fin.wait()
    fout = pltpu.make_async_copy(pbuf, cache_out_a.at[pl.ds(dst_page, 1)], psem)
    fout.start()
    fout.wait()


def kernel(cache, tokens, seq_lens, page_indices, parent_local_id,
           child_local_id, fresh_page):
    meta = jnp.stack([jnp.asarray(parent_local_id, jnp.int32),
                      jnp.asarray(child_local_id, jnp.int32),
                      jnp.asarray(fresh_page, jnp.int32)])
    out = pl.pallas_call(
        _body,
        in_specs=[
            pl.BlockSpec(memory_space=pltpu.SMEM),
            pl.BlockSpec(memory_space=pltpu.SMEM),
            pl.BlockSpec(memory_space=pltpu.SMEM),
            pl.BlockSpec(memory_space=pltpu.VMEM),
            pl.BlockSpec(memory_space=pltpu.VMEM),
            pl.BlockSpec(memory_space=pl.ANY),
        ],
        out_specs=[
            pl.BlockSpec(memory_space=pl.ANY),
            pl.BlockSpec(memory_space=pltpu.VMEM),
            pl.BlockSpec(memory_space=pltpu.SMEM),
            pl.BlockSpec(memory_space=pltpu.VMEM),
        ],
        out_shape=[
            jax.ShapeDtypeStruct(cache.shape, cache.dtype),
            jax.ShapeDtypeStruct(tokens.shape, tokens.dtype),
            jax.ShapeDtypeStruct(seq_lens.shape, seq_lens.dtype),
            jax.ShapeDtypeStruct(page_indices.shape, page_indices.dtype),
        ],
        scratch_shapes=[
            pltpu.VMEM((_K, _BP, PAGE_SIZE, KV_DIM), jnp.float32),
            pltpu.VMEM((1, PAGE_SIZE, KV_DIM), jnp.float32),
            pltpu.SemaphoreType.DMA((_K,)),
            pltpu.SemaphoreType.DMA((_K,)),
            pltpu.SemaphoreType.DMA,
        ],
        compiler_params=pltpu.CompilerParams(
            vmem_limit_bytes=128 * 1024 * 1024),
    )(meta, seq_lens, page_indices, page_indices, tokens, cache)
    cache_out, tokens_out, seq_lens_out, pi_out = out
    return (cache_out, tokens_out, seq_lens_out, pi_out)
